# baseline (speedup 1.0000x reference)
import functools

import jax
import jax.numpy as jnp
from jax import lax
from jax.experimental import pallas as pl
from jax.experimental.pallas import tpu as pltpu
from jax.experimental.pallas import tpu_sc as plsc

_CHUNK = 128


def _make_sc_gather(B, F, n_cores, n_subcores):
  nw = n_cores * n_subcores
  b_per_w = B // nw
  n_chunks = b_per_w // _CHUNK
  mesh = plsc.VectorSubcoreMesh(core_axis_name="c", subcore_axis_name="s")

  f32 = jnp.float32
  out_t = jax.ShapeDtypeStruct((B, 2 * F), f32)

  @functools.partial(
      pl.kernel,
      mesh=mesh,
      compiler_params=pltpu.CompilerParams(use_tc_tiling_on_sc=False),
      out_type=[out_t, out_t],
      scratch_types=[
          pltpu.VMEM((n_chunks, _CHUNK), jnp.int32),
          pltpu.VMEM((n_chunks, _CHUNK), jnp.int32),
          pltpu.VMEM((_CHUNK, F), f32),
          pltpu.VMEM((_CHUNK, F), f32),
          pltpu.SemaphoreType.DMA,
          pltpu.SemaphoreType.DMA,
          pltpu.SemaphoreType.DMA,
          pltpu.SemaphoreType.DMA,
      ],
  )
  def sc_gather(uid_hbm, iid_hbm, pt_hbm, qt_hbm, ut_hbm, vt_hbm,
                mf_out, mlp_out,
                uidx, iidx, buf0, buf1, gsem0, gsem1, wsem0, wsem1):
    wid = lax.axis_index("s") * n_cores + lax.axis_index("c")
    base = wid * b_per_w
    for j in range(n_chunks):
      pltpu.sync_copy(uid_hbm.at[pl.ds(base + j * _CHUNK, _CHUNK)], uidx.at[j])
      pltpu.sync_copy(iid_hbm.at[pl.ds(base + j * _CHUNK, _CHUNK)], iidx.at[j])
    for idx in (uidx, iidx):
      for j in range(n_chunks):
        row = idx.at[j]
        for m in range(_CHUNK // 16):
          v = row[pl.ds(m * 16, 16)]
          row[pl.ds(m * 16, 16)] = (
              (v & -256) | ((v & 127) << 1) | ((v >> 7) & 1))

    ops = []
    for tbl, idx, out, col in ((pt_hbm, uidx, mf_out, 0),
                               (qt_hbm, iidx, mf_out, F),
                               (ut_hbm, uidx, mlp_out, 0),
                               (vt_hbm, iidx, mlp_out, F)):
      for j in range(n_chunks):
        ops.append((tbl, idx.at[j],
                    out.at[pl.ds(base + j * _CHUNK, _CHUNK), pl.ds(col, F)]))

    bufs = (buf0, buf1)
    gsems = (gsem0, gsem1)
    wsems = (wsem0, wsem1)
    gath = [None, None]
    wrt = [None, None]
    for k in range(len(ops) + 1):
      if k < len(ops):
        tbl, idx_ref, _ = ops[k]
        slot = k % 2
        if wrt[slot] is not None:
          wrt[slot].wait()
        gath[slot] = pltpu.async_copy(tbl.at[idx_ref], bufs[slot], gsems[slot])
      if k >= 1:
        pslot = (k - 1) % 2
        _, _, dst = ops[k - 1]
        gath[pslot].wait()
        wrt[pslot] = pltpu.async_copy(bufs[pslot], dst, wsems[pslot])
    for slot in (0, 1):
      if wrt[slot] is not None:
        wrt[slot].wait()

  return sc_gather


def _transpose_body(pts, qts, uts, vts, po, qo, uo, vo):
  ngroups = po.shape[0] // 128
  for src, dst in ((pts, po), (qts, qo), (uts, uo), (vts, vo)):
    x = src[...]
    for g in range(ngroups):
      dst[g * 128:(g + 1) * 128, 0:64] = x[:, 2 * g * 128:(2 * g + 1) * 128].T
      dst[g * 128:(g + 1) * 128, 64:128] = (
          x[:, (2 * g + 1) * 128:(2 * g + 2) * 128].T)


def _relayout_tables(Pt, Qt, Ut, Vt):
  N, F = Pt.shape
  w = 2048
  nblocks = pl.cdiv(N // 2, w)
  npad = nblocks * 2 * w
  in_spec = pl.BlockSpec((F, 2 * w), lambda i: (0, i))
  out_spec = pl.BlockSpec((w, 2 * F), lambda i: (i, 0))
  out_t = jax.ShapeDtypeStruct((npad // 2, 2 * F), jnp.float32)
  outs = pl.pallas_call(
      _transpose_body,
      grid=(nblocks,),
      in_specs=[in_spec] * 4,
      out_specs=[out_spec] * 4,
      out_shape=[out_t] * 4,
  )(jnp.swapaxes(Pt, 0, 1), jnp.swapaxes(Qt, 0, 1),
    jnp.swapaxes(Ut, 0, 1), jnp.swapaxes(Vt, 0, 1))
  return tuple(o.reshape(npad, F) for o in outs)


def _mlp_body(mf, mlpin, w1, b1, w2, b2, w3, b3, wpg, wph, bp, out):
  f32 = jnp.float32
  F = mf.shape[1] // 2
  h = jnp.dot(mlpin[...], w1[...], preferred_element_type=f32)
  h = jnp.maximum(h + b1[...], 0.0)
  h = jnp.maximum(jnp.dot(h, w2[...], preferred_element_type=f32) + b2[...], 0.0)
  h = jnp.maximum(jnp.dot(h, w3[...], preferred_element_type=f32) + b3[...], 0.0)
  m = mf[...]
  g = m[:, :F] * m[:, F:]
  pred = (jnp.sum(g * wpg[...], axis=1) + jnp.sum(h * wph[...], axis=1)
          + bp[0, 0])
  out[...] = 1.0 / (1.0 + jnp.exp(-pred))


def _run_mlp(mf, mlpin, W1, b1, W2, b2, W3, b3, Wp, bp, bs):
  B = mf.shape[0]
  F = mf.shape[1] // 2
  H1 = W1.shape[1]
  H2 = W2.shape[1]
  H3 = W3.shape[1]
  grid = (B // bs,)
  row_spec = pl.BlockSpec((bs, 2 * F), lambda i: (i, 0))
  full = lambda r, c: pl.BlockSpec((r, c), lambda i: (0, 0))
  out = pl.pallas_call(
      _mlp_body,
      grid=grid,
      in_specs=[
          row_spec, row_spec,
          full(2 * F, H1), full(1, H1),
          full(H1, H2), full(1, H2),
          full(H2, H3), full(1, H3),
          full(1, F), full(1, H3), full(1, 1),
      ],
      out_specs=pl.BlockSpec((bs,), lambda i: (i,)),
      out_shape=jax.ShapeDtypeStruct((B,), jnp.float32),
  )(mf, mlpin,
    W1, b1.reshape(1, H1),
    W2, b2.reshape(1, H2),
    W3, b3.reshape(1, H3),
    Wp[:F].reshape(1, F), Wp[F:].reshape(1, H3), bp.reshape(1, 1))
  return out


def kernel(user_id, item_id, Pt, Qt, Ut, Vt, W1, b1, W2, b2, W3, b3, Wp, bp):
  B = user_id.shape[0]
  F = Pt.shape[1]
  info = plsc.get_sparse_core_info()
  sc_gather = _make_sc_gather(B, F, info.num_cores, info.num_subcores)
  uid = user_id.astype(jnp.int32)
  iid = item_id.astype(jnp.int32)
  Ptc, Qtc, Utc, Vtc = _relayout_tables(Pt, Qt, Ut, Vt)
  mf, mlpin = sc_gather(uid, iid, Ptc, Qtc, Utc, Vtc)
  out = _run_mlp(mf, mlpin, W1, b1, W2, b2, W3, b3, Wp, bp, bs=8192)
  return out.reshape(B, 1)


# speedup vs baseline: 1.0173x; 1.0173x over previous
import functools

import jax
import jax.numpy as jnp
from jax import lax
from jax.experimental import pallas as pl
from jax.experimental.pallas import tpu as pltpu
from jax.experimental.pallas import tpu_sc as plsc

_CHUNK = 128


def _make_sc_gather(B, F, n_cores, n_subcores):
  nw = n_cores * n_subcores
  b_per_w = B // nw
  n_chunks = b_per_w // _CHUNK
  mesh = plsc.VectorSubcoreMesh(core_axis_name="c", subcore_axis_name="s")

  f32 = jnp.float32
  out_t = jax.ShapeDtypeStruct((B, 2 * F), f32)

  @functools.partial(
      pl.kernel,
      mesh=mesh,
      compiler_params=pltpu.CompilerParams(use_tc_tiling_on_sc=False),
      out_type=[out_t, out_t],
      scratch_types=[
          pltpu.VMEM((n_chunks, _CHUNK), jnp.int32),
          pltpu.VMEM((n_chunks, _CHUNK), jnp.int32),
          pltpu.VMEM((_CHUNK, F), f32),
          pltpu.VMEM((_CHUNK, F), f32),
          pltpu.VMEM((_CHUNK, F), f32),
          pltpu.VMEM((_CHUNK, F), f32),
      ] + [pltpu.SemaphoreType.DMA] * 8,
  )
  def sc_gather(uid_hbm, iid_hbm, pt_hbm, qt_hbm, ut_hbm, vt_hbm,
                mf_out, mlp_out,
                uidx, iidx, buf0, buf1, buf2, buf3,
                gsem0, gsem1, gsem2, gsem3, wsem0, wsem1, wsem2, wsem3):
    wid = lax.axis_index("s") * n_cores + lax.axis_index("c")
    base = wid * b_per_w
    for j in range(n_chunks):
      pltpu.sync_copy(uid_hbm.at[pl.ds(base + j * _CHUNK, _CHUNK)], uidx.at[j])
      pltpu.sync_copy(iid_hbm.at[pl.ds(base + j * _CHUNK, _CHUNK)], iidx.at[j])
    for idx in (uidx, iidx):
      for j in range(n_chunks):
        row = idx.at[j]
        for m in range(_CHUNK // 16):
          v = row[pl.ds(m * 16, 16)]
          row[pl.ds(m * 16, 16)] = (
              (v & -256) | ((v & 127) << 1) | ((v >> 7) & 1))

    ops = []
    for tbl, idx, out, col in ((pt_hbm, uidx, mf_out, 0),
                               (qt_hbm, iidx, mf_out, F),
                               (ut_hbm, uidx, mlp_out, 0),
                               (vt_hbm, iidx, mlp_out, F)):
      for j in range(n_chunks):
        ops.append((tbl, idx.at[j],
                    out.at[pl.ds(base + j * _CHUNK, _CHUNK), pl.ds(col, F)]))

    bufs = (buf0, buf1, buf2, buf3)
    gsems = (gsem0, gsem1, gsem2, gsem3)
    wsems = (wsem0, wsem1, wsem2, wsem3)
    nslots = 4
    gath = [None] * nslots
    wrt = [None] * nslots
    for k in range(len(ops) + 1):
      if k < len(ops):
        tbl, idx_ref, _ = ops[k]
        slot = k % nslots
        if wrt[slot] is not None:
          wrt[slot].wait()
        gath[slot] = pltpu.async_copy(tbl.at[idx_ref], bufs[slot], gsems[slot])
      if k >= 1:
        pslot = (k - 1) % nslots
        _, _, dst = ops[k - 1]
        gath[pslot].wait()
        wrt[pslot] = pltpu.async_copy(bufs[pslot], dst, wsems[pslot])
    for slot in range(nslots):
      if wrt[slot] is not None:
        wrt[slot].wait()

  return sc_gather


def _transpose_body(pts, qts, uts, vts, po, qo, uo, vo):
  ngroups = po.shape[0] // 128
  for src, dst in ((pts, po), (qts, qo), (uts, uo), (vts, vo)):
    x = src[...]
    for g in range(ngroups):
      dst[g * 128:(g + 1) * 128, 0:64] = x[:, 2 * g * 128:(2 * g + 1) * 128].T
      dst[g * 128:(g + 1) * 128, 64:128] = (
          x[:, (2 * g + 1) * 128:(2 * g + 2) * 128].T)


def _relayout_tables(Pt, Qt, Ut, Vt):
  N, F = Pt.shape
  w = 2048
  nblocks = pl.cdiv(N // 2, w)
  npad = nblocks * 2 * w
  in_spec = pl.BlockSpec((F, 2 * w), lambda i: (0, i))
  out_spec = pl.BlockSpec((w, 2 * F), lambda i: (i, 0))
  out_t = jax.ShapeDtypeStruct((npad // 2, 2 * F), jnp.float32)
  outs = pl.pallas_call(
      _transpose_body,
      grid=(nblocks,),
      in_specs=[in_spec] * 4,
      out_specs=[out_spec] * 4,
      out_shape=[out_t] * 4,
  )(jnp.swapaxes(Pt, 0, 1), jnp.swapaxes(Qt, 0, 1),
    jnp.swapaxes(Ut, 0, 1), jnp.swapaxes(Vt, 0, 1))
  return tuple(o.reshape(npad, F) for o in outs)


def _mlp_body(mf, mlpin, w1, b1, w2, b2, w3, b3, wpg, wph, bp, out):
  f32 = jnp.float32
  F = mf.shape[1] // 2
  h = jnp.dot(mlpin[...], w1[...], preferred_element_type=f32)
  h = jnp.maximum(h + b1[...], 0.0)
  h = jnp.maximum(jnp.dot(h, w2[...], preferred_element_type=f32) + b2[...], 0.0)
  h = jnp.maximum(jnp.dot(h, w3[...], preferred_element_type=f32) + b3[...], 0.0)
  m = mf[...]
  g = m[:, :F] * m[:, F:]
  pred = (jnp.sum(g * wpg[...], axis=1) + jnp.sum(h * wph[...], axis=1)
          + bp[0, 0])
  out[...] = 1.0 / (1.0 + jnp.exp(-pred))


def _run_mlp(mf, mlpin, W1, b1, W2, b2, W3, b3, Wp, bp, bs):
  B = mf.shape[0]
  F = mf.shape[1] // 2
  H1 = W1.shape[1]
  H2 = W2.shape[1]
  H3 = W3.shape[1]
  grid = (B // bs,)
  row_spec = pl.BlockSpec((bs, 2 * F), lambda i: (i, 0))
  full = lambda r, c: pl.BlockSpec((r, c), lambda i: (0, 0))
  out = pl.pallas_call(
      _mlp_body,
      grid=grid,
      in_specs=[
          row_spec, row_spec,
          full(2 * F, H1), full(1, H1),
          full(H1, H2), full(1, H2),
          full(H2, H3), full(1, H3),
          full(1, F), full(1, H3), full(1, 1),
      ],
      out_specs=pl.BlockSpec((bs,), lambda i: (i,)),
      out_shape=jax.ShapeDtypeStruct((B,), jnp.float32),
  )(mf, mlpin,
    W1, b1.reshape(1, H1),
    W2, b2.reshape(1, H2),
    W3, b3.reshape(1, H3),
    Wp[:F].reshape(1, F), Wp[F:].reshape(1, H3), bp.reshape(1, 1))
  return out


def kernel(user_id, item_id, Pt, Qt, Ut, Vt, W1, b1, W2, b2, W3, b3, Wp, bp):
  B = user_id.shape[0]
  F = Pt.shape[1]
  info = plsc.get_sparse_core_info()
  sc_gather = _make_sc_gather(B, F, info.num_cores, info.num_subcores)
  uid = user_id.astype(jnp.int32)
  iid = item_id.astype(jnp.int32)
  Ptc, Qtc, Utc, Vtc = _relayout_tables(Pt, Qt, Ut, Vt)
  mf, mlpin = sc_gather(uid, iid, Ptc, Qtc, Utc, Vtc)
  out = _run_mlp(mf, mlpin, W1, b1, W2, b2, W3, b3, Wp, bp, bs=4096)
  return out.reshape(B, 1)
